# baseline (speedup 1.0000x reference)
import functools

import jax
import jax.numpy as jnp
from jax import lax
from jax.experimental import pallas as pl
from jax.experimental.pallas import tpu as pltpu
from jax.experimental.pallas import tpu_sc as plsc

HID = 768
B = 1024
L = 128
EPS = 1e-12
LANES = 16
NC = 2
NS = 16
NW = NC * NS
TPW = L // NW
CHUNK = 16
NCHUNK = B // CHUNK
NCHUNK_SHIFT = NCHUNK.bit_length() - 1
TOTAL = TPW * NCHUNK
NBUF = 4
BPW_MASK = B // NW
VECS = HID // LANES
OUT_COLS = L + L * HID

_GATHER_DNUMS = lax.GatherDimensionNumbers(
    offset_dims=(), collapsed_slice_dims=(0,), start_index_map=(0,))


@functools.partial(
    pl.kernel,
    out_type=jax.ShapeDtypeStruct((B, OUT_COLS), jnp.float32),
    mesh=plsc.VectorSubcoreMesh(core_axis_name="c", subcore_axis_name="s"),
    compiler_params=pltpu.CompilerParams(needs_layout_passes=False),
    scratch_types=[
        pltpu.VMEM((TOTAL, CHUNK), jnp.int32),
        pltpu.VMEM((CHUNK, HID), jnp.float32),
        pltpu.VMEM((CHUNK, HID), jnp.float32),
        pltpu.VMEM((CHUNK, HID), jnp.float32),
        pltpu.VMEM((CHUNK, HID), jnp.float32),
        pltpu.VMEM((TPW, HID), jnp.float32),
        pltpu.VMEM((HID,), jnp.float32),
        pltpu.VMEM((HID,), jnp.float32),
        pltpu.VMEM((CHUNK, LANES), jnp.float32),
        pltpu.VMEM((CHUNK, LANES), jnp.float32),
        pltpu.SMEM((CHUNK,), jnp.float32),
        pltpu.SMEM((CHUNK,), jnp.float32),
        pltpu.VMEM((BPW_MASK, L), jnp.int32),
        pltpu.VMEM((BPW_MASK, L), jnp.float32),
        pltpu.SemaphoreType.DMA,
        pltpu.SemaphoreType.DMA,
        pltpu.SemaphoreType.DMA,
        pltpu.SemaphoreType.DMA,
        pltpu.SemaphoreType.DMA,
        pltpu.SemaphoreType.DMA,
        pltpu.SemaphoreType.DMA,
        pltpu.SemaphoreType.DMA,
    ],
)
def _embed_ln(ids_t_hbm, mask_hbm, word_hbm, base_hbm, gamma_hbm, beta_hbm,
              out_hbm, idx_v, rows0, rows1, rows2, rows3, base_v, g_v, b_v,
              st_s, st_q, av_v, cv_v, mi_v, mf_v,
              gs0, gs1, gs2, gs3, ss0, ss1, ss2, ss3):
    cid = lax.axis_index("c")
    sid = lax.axis_index("s")
    w = sid * NC + cid
    iota = jnp.arange(LANES, dtype=jnp.int32)
    ROWS = [rows0, rows1, rows2, rows3]
    GS = [gs0, gs1, gs2, gs3]
    SS = [ss0, ss1, ss2, ss3]

    mb0 = w * BPW_MASK
    pltpu.sync_copy(mask_hbm.at[pl.ds(mb0, BPW_MASK)], mi_v)

    @plsc.parallel_loop(0, BPW_MASK, 1)
    def _mask_row(r):
        for c in range(L // LANES):
            sl = pl.ds(c * LANES, LANES)
            mf_v[r, sl] = mi_v[r, sl].astype(jnp.float32)
    pltpu.sync_copy(mf_v, out_hbm.at[pl.ds(mb0, BPW_MASK), pl.ds(0, L)])

    pltpu.sync_copy(gamma_hbm, g_v)
    pltpu.sync_copy(beta_hbm, b_v)
    pltpu.sync_copy(ids_t_hbm.at[pl.ds(w * TOTAL, TOTAL)], idx_v)
    pltpu.sync_copy(base_hbm.at[pl.ds(w * TPW, TPW)], base_v)

    def out_slab(j):
        t_loc = j >> NCHUNK_SHIFT
        ck = j & (NCHUNK - 1)
        col0 = L + (w * TPW + t_loc) * HID
        return out_hbm.at[pl.ds(ck * CHUNK, CHUNK), pl.ds(col0, HID)]

    pltpu.async_copy(rows3, out_slab(TOTAL - 1), SS[3])
    for p in range(NBUF - 1):
        pltpu.async_copy(word_hbm.at[idx_v.at[p]], ROWS[p], GS[p])

    def compute_chunk(rows_v, t_loc):
        def p1_row(r, cr):
            z = jnp.zeros((LANES,), jnp.float32)

            def _elem(c, acc):
                a0, q0 = acc
                sl = pl.ds(pl.multiple_of(c * LANES, LANES), LANES)
                x = rows_v[r, sl] + base_v[t_loc, sl]
                return (a0 + x, q0 + x * x)

            a0, q0 = plsc.parallel_loop(
                0, VECS, 1, unroll=8, carry=(z, z))(_elem)
            st_s[r, :] = a0
            st_q[r, :] = q0
            return cr

        lax.fori_loop(0, CHUNK, p1_row, 0)

        @plsc.parallel_loop(0, CHUNK // LANES, 1)
        def _stats(g):
            r0 = g * LANES
            ridx = r0 + iota
            ssum = jnp.zeros((LANES,), jnp.float32)
            qsum = jnp.zeros((LANES,), jnp.float32)
            for c in range(LANES):
                cc = jnp.full((LANES,), c, jnp.int32)
                ssum = ssum + plsc.load_gather(st_s, [ridx, cc])
                qsum = qsum + plsc.load_gather(st_q, [ridx, cc])
            m = ssum * (1.0 / HID)
            q = qsum * (1.0 / HID)
            v = q - m * m + EPS
            iv = lax.bitcast_convert_type(v, jnp.int32)
            iv = jnp.int32(0x5F3759DF) - lax.shift_right_logical(iv, 1)
            y = lax.bitcast_convert_type(iv, jnp.float32)
            for _ in range(3):
                y = y * (1.5 - 0.5 * v * y * y)
            cshift = 0.0 - m * y
            for k in range(LANES):
                av_v[r0 + k] = y[k]
                cv_v[r0 + k] = cshift[k]

        def p2_cb(cb, cc2):
            sl = pl.ds(pl.multiple_of(cb * LANES, LANES), LANES)
            bc = base_v[t_loc, sl]
            gv = g_v[sl]
            bv = b_v[sl]

            @plsc.parallel_loop(0, CHUNK, 1, unroll=8)
            def _p2_row(r):
                a = av_v[r]
                c0 = cv_v[r]
                x = rows_v[r, sl] + bc
                tt = x * a + c0
                rows_v[r, sl] = tt * gv + bv

            return cc2

        lax.fori_loop(0, VECS, p2_cb, 0)

    def outer(g, carry):
        for i in range(NBUF):
            j = g * NBUF + i
            b = i
            bp = (b + NBUF - 1) % NBUF
            pltpu.make_async_copy(
                word_hbm.at[idx_v.at[j]], ROWS[b], GS[b]).wait()
            compute_chunk(ROWS[b], j >> NCHUNK_SHIFT)
            pltpu.async_copy(ROWS[b], out_slab(j), SS[b])
            jm = (j + TOTAL - 1) & (TOTAL - 1)
            pltpu.make_async_copy(ROWS[bp], out_slab(jm), SS[bp]).wait()
            jn = jnp.minimum(j + NBUF - 1, TOTAL - 1)
            pltpu.async_copy(word_hbm.at[idx_v.at[jn]], ROWS[bp], GS[bp])
        return carry

    lax.fori_loop(0, TOTAL // NBUF, outer, 0)

    pltpu.make_async_copy(ROWS[3], out_slab(TOTAL - 1), SS[3]).wait()
    for p in range(NBUF - 1):
        pltpu.make_async_copy(
            word_hbm.at[idx_v.at[TOTAL - 1]], ROWS[p], GS[p]).wait()


def kernel(input, word_embeddings, position_embeddings, token_type_embeddings,
           ln_gamma, ln_beta):
    ids = input[:, 0, :].astype(jnp.int32)
    mask = input[:, 1, :].astype(jnp.int32)
    ids_t = ids.T.reshape(L * NCHUNK, CHUNK)
    base = position_embeddings[:L] + token_type_embeddings[0][None, :]
    return _embed_ln(ids_t, mask, word_embeddings, base, ln_gamma, ln_beta)


# speedup vs baseline: 1.1529x; 1.1529x over previous
import functools

import jax
import jax.numpy as jnp
from jax import lax
from jax.experimental import pallas as pl
from jax.experimental.pallas import tpu as pltpu
from jax.experimental.pallas import tpu_sc as plsc

HID = 768
B = 1024
L = 128
EPS = 1e-12
LANES = 16
NC = 2
NS = 16
NW = NC * NS
TPW = L // NW
CHUNK = 32
MROWS = 8
NCHUNK = B // CHUNK
NCHUNK_SHIFT = NCHUNK.bit_length() - 1
TOTAL = TPW * NCHUNK
NBUF = 4
BPW_MASK = B // NW
VECS = HID // LANES
OUT_COLS = L + L * HID

_GATHER_DNUMS = lax.GatherDimensionNumbers(
    offset_dims=(), collapsed_slice_dims=(0,), start_index_map=(0,))


@functools.partial(
    pl.kernel,
    out_type=jax.ShapeDtypeStruct((B, OUT_COLS), jnp.float32),
    mesh=plsc.VectorSubcoreMesh(core_axis_name="c", subcore_axis_name="s"),
    compiler_params=pltpu.CompilerParams(needs_layout_passes=False),
    scratch_types=[
        pltpu.VMEM((TOTAL, CHUNK), jnp.int32),
        pltpu.VMEM((CHUNK, HID), jnp.float32),
        pltpu.VMEM((CHUNK, HID), jnp.float32),
        pltpu.VMEM((CHUNK, HID), jnp.float32),
        pltpu.VMEM((CHUNK, HID), jnp.float32),
        pltpu.VMEM((TPW, HID), jnp.float32),
        pltpu.VMEM((HID,), jnp.float32),
        pltpu.VMEM((HID,), jnp.float32),
        pltpu.VMEM((CHUNK, LANES), jnp.float32),
        pltpu.VMEM((CHUNK, LANES), jnp.float32),
        pltpu.SMEM((CHUNK,), jnp.float32),
        pltpu.SMEM((CHUNK,), jnp.float32),
        pltpu.VMEM((MROWS, L), jnp.int32),
        pltpu.VMEM((MROWS, L), jnp.float32),
        pltpu.SemaphoreType.DMA,
        pltpu.SemaphoreType.DMA,
        pltpu.SemaphoreType.DMA,
        pltpu.SemaphoreType.DMA,
        pltpu.SemaphoreType.DMA,
        pltpu.SemaphoreType.DMA,
        pltpu.SemaphoreType.DMA,
        pltpu.SemaphoreType.DMA,
    ],
)
def _embed_ln(ids_t_hbm, mask_hbm, word_hbm, base_hbm, gamma_hbm, beta_hbm,
              out_hbm, idx_v, rows0, rows1, rows2, rows3, base_v, g_v, b_v,
              st_s, st_q, av_v, cv_v, mi_v, mf_v,
              gs0, gs1, gs2, gs3, ss0, ss1, ss2, ss3):
    cid = lax.axis_index("c")
    sid = lax.axis_index("s")
    w = sid * NC + cid
    iota = jnp.arange(LANES, dtype=jnp.int32)
    ROWS = [rows0, rows1, rows2, rows3]
    GS = [gs0, gs1, gs2, gs3]
    SS = [ss0, ss1, ss2, ss3]

    mb0 = w * BPW_MASK
    for h in range(BPW_MASK // MROWS):
        pltpu.sync_copy(mask_hbm.at[pl.ds(mb0 + h * MROWS, MROWS)], mi_v)

        @plsc.parallel_loop(0, MROWS, 1)
        def _mask_row(r):
            for c in range(L // LANES):
                sl = pl.ds(c * LANES, LANES)
                mf_v[r, sl] = mi_v[r, sl].astype(jnp.float32)
        pltpu.sync_copy(
            mf_v, out_hbm.at[pl.ds(mb0 + h * MROWS, MROWS), pl.ds(0, L)])

    pltpu.sync_copy(gamma_hbm, g_v)
    pltpu.sync_copy(beta_hbm, b_v)
    pltpu.sync_copy(ids_t_hbm.at[pl.ds(w * TOTAL, TOTAL)], idx_v)
    pltpu.sync_copy(base_hbm.at[pl.ds(w * TPW, TPW)], base_v)

    def out_slab(j):
        t_loc = j >> NCHUNK_SHIFT
        ck = j & (NCHUNK - 1)
        col0 = L + (w * TPW + t_loc) * HID
        return out_hbm.at[pl.ds(ck * CHUNK, CHUNK), pl.ds(col0, HID)]

    pltpu.async_copy(rows3, out_slab(TOTAL - 1), SS[3])
    for p in range(NBUF - 1):
        pltpu.async_copy(word_hbm.at[idx_v.at[p]], ROWS[p], GS[p])

    def compute_chunk(rows_v, t_loc):
        def p1_row(r, cr):
            z = jnp.zeros((LANES,), jnp.float32)

            def _elem(c, acc):
                a0, q0 = acc
                sl = pl.ds(pl.multiple_of(c * LANES, LANES), LANES)
                x = rows_v[r, sl] + base_v[t_loc, sl]
                return (a0 + x, q0 + x * x)

            a0, q0 = plsc.parallel_loop(
                0, VECS, 1, unroll=4, carry=(z, z))(_elem)
            st_s[r, :] = a0
            st_q[r, :] = q0
            return cr

        lax.fori_loop(0, CHUNK, p1_row, 0)

        @plsc.parallel_loop(0, CHUNK // LANES, 1)
        def _stats(g):
            r0 = g * LANES
            ridx = r0 + iota
            ssum = jnp.zeros((LANES,), jnp.float32)
            qsum = jnp.zeros((LANES,), jnp.float32)
            for c in range(LANES):
                cc = jnp.full((LANES,), c, jnp.int32)
                ssum = ssum + plsc.load_gather(st_s, [ridx, cc])
                qsum = qsum + plsc.load_gather(st_q, [ridx, cc])
            m = ssum * (1.0 / HID)
            q = qsum * (1.0 / HID)
            v = q - m * m + EPS
            iv = lax.bitcast_convert_type(v, jnp.int32)
            iv = jnp.int32(0x5F3759DF) - lax.shift_right_logical(iv, 1)
            y = lax.bitcast_convert_type(iv, jnp.float32)
            for _ in range(3):
                y = y * (1.5 - 0.5 * v * y * y)
            cshift = 0.0 - m * y
            for k in range(LANES):
                av_v[r0 + k] = y[k]
                cv_v[r0 + k] = cshift[k]

        def p2_cb(cb, cc2):
            sl = pl.ds(pl.multiple_of(cb * LANES, LANES), LANES)
            bc = base_v[t_loc, sl]
            gv = g_v[sl]
            bv = b_v[sl]

            @plsc.parallel_loop(0, CHUNK, 1, unroll=4)
            def _p2_row(r):
                a = av_v[r]
                c0 = cv_v[r]
                x = rows_v[r, sl] + bc
                tt = x * a + c0
                rows_v[r, sl] = tt * gv + bv

            return cc2

        lax.fori_loop(0, VECS, p2_cb, 0)

    def outer(g, carry):
        for i in range(NBUF):
            j = g * NBUF + i
            b = i
            bp = (b + NBUF - 1) % NBUF
            pltpu.make_async_copy(
                word_hbm.at[idx_v.at[j]], ROWS[b], GS[b]).wait()
            compute_chunk(ROWS[b], j >> NCHUNK_SHIFT)
            pltpu.async_copy(ROWS[b], out_slab(j), SS[b])
            jm = (j + TOTAL - 1) & (TOTAL - 1)
            pltpu.make_async_copy(ROWS[bp], out_slab(jm), SS[bp]).wait()
            jn = jnp.minimum(j + NBUF - 1, TOTAL - 1)
            pltpu.async_copy(word_hbm.at[idx_v.at[jn]], ROWS[bp], GS[bp])
        return carry

    lax.fori_loop(0, TOTAL // NBUF, outer, 0)

    pltpu.make_async_copy(ROWS[3], out_slab(TOTAL - 1), SS[3]).wait()
    for p in range(NBUF - 1):
        pltpu.make_async_copy(
            word_hbm.at[idx_v.at[TOTAL - 1]], ROWS[p], GS[p]).wait()


def kernel(input, word_embeddings, position_embeddings, token_type_embeddings,
           ln_gamma, ln_beta):
    ids = input[:, 0, :].astype(jnp.int32)
    mask = input[:, 1, :].astype(jnp.int32)
    ids_t = ids.T.reshape(L * NCHUNK, CHUNK)
    base = position_embeddings[:L] + token_type_embeddings[0][None, :]
    return _embed_ln(ids_t, mask, word_embeddings, base, ln_gamma, ln_beta)
